# baseline (speedup 1.0000x reference)
import functools

import jax
import jax.numpy as jnp
from jax import lax
from jax.experimental import pallas as pl
from jax.experimental.pallas import tpu as pltpu
from jax.experimental.pallas import tpu_sc as plsc

_CHUNK = 256
_LANES = 16


def _sc_lengths(mt_flat, S, B):
    mesh = plsc.VectorSubcoreMesh(core_axis_name="c", subcore_axis_name="s")

    @functools.partial(
        pl.kernel,
        mesh=mesh,
        out_type=jax.ShapeDtypeStruct((B,), jnp.int32),
        scratch_types=[
            pltpu.VMEM((S * B,), jnp.int32),
            pltpu.VMEM((B,), jnp.int32),
            pltpu.SemaphoreType.DMA,
        ],
    )
    def k(m_hbm, out_hbm, m_v, len_v, sem):
        wid = lax.axis_index("s") * 2 + lax.axis_index("c")

        @pl.when(wid == 0)
        def _():
            pltpu.async_copy(m_hbm, m_v, sem).wait()

            def add_row(s, acc):
                return acc + m_v[pl.ds(s * B, B)]

            len_v[...] = lax.fori_loop(
                0, S, add_row, jnp.zeros((B,), jnp.int32), unroll=8
            )
            pltpu.async_copy(len_v, out_hbm, sem).wait()

    return k(mt_flat)


def _body(l_ref, m_ref, x_hbm, o_ref, scratch, sems):
    b = pl.program_id(0)
    nb = pl.num_programs(0)
    _, S, D = scratch.shape
    nc = S // _CHUNK

    def chunk_copy(row, buf, c):
        return pltpu.make_async_copy(
            x_hbm.at[row, pl.ds(c * _CHUNK, _CHUNK), :],
            scratch.at[buf, pl.ds(c * _CHUNK, _CHUNK), :],
            sems.at[buf],
        )

    def issue(row, buf):
        nch = (l_ref[row] + _CHUNK - 1) // _CHUNK

        def st(c, carry):
            @pl.when(c < nch)
            def _():
                chunk_copy(row, buf, c).start()
            return carry

        jax.lax.fori_loop(0, nc, st, 0, unroll=True)

    def wait_row(row, buf):
        nch = (l_ref[row] + _CHUNK - 1) // _CHUNK

        def wt(c, carry):
            @pl.when(c < nch)
            def _():
                chunk_copy(row, buf, c).wait()
            return carry

        jax.lax.fori_loop(0, nc, wt, 0, unroll=True)

    @pl.when(b == 0)
    def _():
        issue(b, 0)

    nxt = b + 1

    @pl.when((nxt < nb) & (nxt % 2 == 0))
    def _():
        issue(nxt, 0)

    @pl.when((nxt < nb) & (nxt % 2 == 1))
    def _():
        issue(nxt, 1)

    @pl.when(b % 2 == 0)
    def _():
        wait_row(b, 0)

    @pl.when(b % 2 == 1)
    def _():
        wait_row(b, 1)

    length = l_ref[b]
    zeros_c = jnp.zeros((_CHUNK, D), dtype=o_ref.dtype)
    for buf in (0, 1):

        @pl.when(b % 2 == buf)
        def _(buf=buf):
            for c in range(nc):
                lo = c * _CHUNK

                @pl.when(lo < length)
                def _(lo=lo):
                    pos = jax.lax.broadcasted_iota(jnp.int32, (_CHUNK, 1), 0) + lo
                    m_t = m_ref[0, 0, pl.ds(lo, _CHUNK)][:, None]
                    keep = (pos < length) & (m_t > 0)
                    o_ref[0, pl.ds(lo, _CHUNK), :] = jnp.where(
                        keep, scratch[buf, pl.ds(lo, _CHUNK), :], zeros_c
                    )

                @pl.when(lo >= length)
                def _(lo=lo):
                    o_ref[0, pl.ds(lo, _CHUNK), :] = zeros_c


def kernel(x, mask):
    B, S, D = x.shape
    mi = mask.astype(jnp.int32)
    lengths = _sc_lengths(mi.T.reshape(-1), S, B)

    m3 = mi.reshape(B, 1, S)
    return pl.pallas_call(
        _body,
        grid=(B,),
        in_specs=[
            pl.BlockSpec(memory_space=pltpu.SMEM),
            pl.BlockSpec((1, 1, S), lambda b: (b, 0, 0)),
            pl.BlockSpec(memory_space=pl.ANY),
        ],
        out_specs=pl.BlockSpec((1, S, D), lambda b: (b, 0, 0)),
        out_shape=jax.ShapeDtypeStruct((B, S, D), x.dtype),
        scratch_shapes=[
            pltpu.VMEM((2, S, D), x.dtype),
            pltpu.SemaphoreType.DMA((2,)),
        ],
    )(lengths, m3, x)


# speedup vs baseline: 1.0073x; 1.0073x over previous
import functools

import jax
import jax.numpy as jnp
from jax import lax
from jax.experimental import pallas as pl
from jax.experimental.pallas import tpu as pltpu
from jax.experimental.pallas import tpu_sc as plsc

_CHUNK = 256
_LANES = 16


def _sc_lengths(mt_flat, S, B):
    mesh = plsc.VectorSubcoreMesh(core_axis_name="c", subcore_axis_name="s")

    NS = 16
    sl = S // NS

    @functools.partial(
        pl.kernel,
        mesh=mesh,
        out_type=jax.ShapeDtypeStruct((B,), jnp.int32),
        scratch_types=[
            pltpu.VMEM((sl * B,), jnp.int32),
            pltpu.VMEM((B,), jnp.int32),
            pltpu.VMEM((B,), jnp.int32),
            pltpu.VMEM_SHARED((NS * B,), jnp.int32),
            pltpu.SemaphoreType.DMA,
        ],
    )
    def k(m_hbm, out_hbm, m_v, acc_v, tmp_v, shared, sem):
        cid = lax.axis_index("c")
        sid = lax.axis_index("s")

        @pl.when(cid == 0)
        def _():
            pltpu.async_copy(m_hbm.at[pl.ds(sid * sl * B, sl * B)], m_v, sem).wait()

            def add_row(s, acc):
                return acc + m_v[pl.ds(s * B, B)]

            acc_v[...] = lax.fori_loop(
                0, sl, add_row, jnp.zeros((B,), jnp.int32), unroll=8
            )
            pltpu.sync_copy(acc_v, shared.at[pl.ds(sid * B, B)])

        plsc.subcore_barrier()

        @pl.when((cid == 0) & (sid == 0))
        def _():
            total = jnp.zeros((B,), jnp.int32)
            for w in range(NS):
                pltpu.sync_copy(shared.at[pl.ds(w * B, B)], tmp_v)
                total = total + tmp_v[...]
            acc_v[...] = total
            pltpu.async_copy(acc_v, out_hbm, sem).wait()

    return k(mt_flat)


def _body(l_ref, m_ref, x_hbm, o_ref, scratch, sems):
    b = pl.program_id(0)
    nb = pl.num_programs(0)
    _, S, D = scratch.shape
    nc = S // _CHUNK

    def chunk_copy(row, buf, c):
        return pltpu.make_async_copy(
            x_hbm.at[row, pl.ds(c * _CHUNK, _CHUNK), :],
            scratch.at[buf, pl.ds(c * _CHUNK, _CHUNK), :],
            sems.at[buf],
        )

    def issue(row, buf):
        nch = (l_ref[row] + _CHUNK - 1) // _CHUNK

        def st(c, carry):
            @pl.when(c < nch)
            def _():
                chunk_copy(row, buf, c).start()
            return carry

        jax.lax.fori_loop(0, nc, st, 0, unroll=True)

    def wait_row(row, buf):
        nch = (l_ref[row] + _CHUNK - 1) // _CHUNK

        def wt(c, carry):
            @pl.when(c < nch)
            def _():
                chunk_copy(row, buf, c).wait()
            return carry

        jax.lax.fori_loop(0, nc, wt, 0, unroll=True)

    @pl.when(b == 0)
    def _():
        issue(b, 0)

    nxt = b + 1

    @pl.when((nxt < nb) & (nxt % 2 == 0))
    def _():
        issue(nxt, 0)

    @pl.when((nxt < nb) & (nxt % 2 == 1))
    def _():
        issue(nxt, 1)

    @pl.when(b % 2 == 0)
    def _():
        wait_row(b, 0)

    @pl.when(b % 2 == 1)
    def _():
        wait_row(b, 1)

    length = l_ref[b]
    zeros_c = jnp.zeros((_CHUNK, D), dtype=o_ref.dtype)
    for buf in (0, 1):

        @pl.when(b % 2 == buf)
        def _(buf=buf):
            for c in range(nc):
                lo = c * _CHUNK

                @pl.when(lo < length)
                def _(lo=lo):
                    pos = jax.lax.broadcasted_iota(jnp.int32, (_CHUNK, 1), 0) + lo
                    m_t = m_ref[0, 0, pl.ds(lo, _CHUNK)][:, None]
                    keep = (pos < length) & (m_t > 0)
                    o_ref[0, pl.ds(lo, _CHUNK), :] = jnp.where(
                        keep, scratch[buf, pl.ds(lo, _CHUNK), :], zeros_c
                    )

                @pl.when(lo >= length)
                def _(lo=lo):
                    o_ref[0, pl.ds(lo, _CHUNK), :] = zeros_c


def kernel(x, mask):
    B, S, D = x.shape
    mi = mask.astype(jnp.int32)
    lengths = _sc_lengths(mi.T.reshape(-1), S, B)

    m3 = mi.reshape(B, 1, S)
    return pl.pallas_call(
        _body,
        grid=(B,),
        in_specs=[
            pl.BlockSpec(memory_space=pltpu.SMEM),
            pl.BlockSpec((1, 1, S), lambda b: (b, 0, 0)),
            pl.BlockSpec(memory_space=pl.ANY),
        ],
        out_specs=pl.BlockSpec((1, S, D), lambda b: (b, 0, 0)),
        out_shape=jax.ShapeDtypeStruct((B, S, D), x.dtype),
        scratch_shapes=[
            pltpu.VMEM((2, S, D), x.dtype),
            pltpu.SemaphoreType.DMA((2,)),
        ],
    )(lengths, m3, x)


# speedup vs baseline: 1.2909x; 1.2815x over previous
import jax
import jax.numpy as jnp
from jax.experimental import pallas as pl
from jax.experimental.pallas import tpu as pltpu

_CHUNK = 256


def _body(m_ref, mn_ref, x_hbm, o_ref, scratch, sems):
    b = pl.program_id(0)
    nb = pl.num_programs(0)
    _, S, D = scratch.shape
    nc = S // _CHUNK

    length = jnp.sum(m_ref[0, 0, :])
    length_nxt = jnp.sum(mn_ref[0, 0, :])

    def chunk_copy(row, buf, c):
        return pltpu.make_async_copy(
            x_hbm.at[row, pl.ds(c * _CHUNK, _CHUNK), :],
            scratch.at[buf, pl.ds(c * _CHUNK, _CHUNK), :],
            sems.at[buf],
        )

    def issue(row, buf, row_len):
        nch = (row_len + _CHUNK - 1) // _CHUNK

        def st(c, carry):
            @pl.when(c < nch)
            def _():
                chunk_copy(row, buf, c).start()
            return carry

        jax.lax.fori_loop(0, nc, st, 0, unroll=True)

    def wait_row(row, buf, row_len):
        nch = (row_len + _CHUNK - 1) // _CHUNK

        def wt(c, carry):
            @pl.when(c < nch)
            def _():
                chunk_copy(row, buf, c).wait()
            return carry

        jax.lax.fori_loop(0, nc, wt, 0, unroll=True)

    @pl.when(b == 0)
    def _():
        issue(b, 0, length)

    nxt = b + 1

    @pl.when((nxt < nb) & (nxt % 2 == 0))
    def _():
        issue(nxt, 0, length_nxt)

    @pl.when((nxt < nb) & (nxt % 2 == 1))
    def _():
        issue(nxt, 1, length_nxt)

    @pl.when(b % 2 == 0)
    def _():
        wait_row(b, 0, length)

    @pl.when(b % 2 == 1)
    def _():
        wait_row(b, 1, length)

    zeros_c = jnp.zeros((_CHUNK, D), dtype=o_ref.dtype)
    for buf in (0, 1):

        @pl.when(b % 2 == buf)
        def _(buf=buf):
            for c in range(nc):
                lo = c * _CHUNK

                @pl.when(lo < length)
                def _(lo=lo):
                    pos = jax.lax.broadcasted_iota(jnp.int32, (_CHUNK, 1), 0) + lo
                    m_t = m_ref[0, 0, pl.ds(lo, _CHUNK)][:, None]
                    keep = (pos < length) & (m_t > 0)
                    o_ref[0, pl.ds(lo, _CHUNK), :] = jnp.where(
                        keep, scratch[buf, pl.ds(lo, _CHUNK), :], zeros_c
                    )

                @pl.when(lo >= length)
                def _(lo=lo):
                    o_ref[0, pl.ds(lo, _CHUNK), :] = zeros_c


def kernel(x, mask):
    B, S, D = x.shape
    m3 = mask.astype(jnp.int32).reshape(B, 1, S)
    return pl.pallas_call(
        _body,
        grid=(B,),
        in_specs=[
            pl.BlockSpec((1, 1, S), lambda b: (b, 0, 0)),
            pl.BlockSpec((1, 1, S), lambda b: (jnp.minimum(b + 1, B - 1), 0, 0)),
            pl.BlockSpec(memory_space=pl.ANY),
        ],
        out_specs=pl.BlockSpec((1, S, D), lambda b: (b, 0, 0)),
        out_shape=jax.ShapeDtypeStruct((B, S, D), x.dtype),
        scratch_shapes=[
            pltpu.VMEM((2, S, D), x.dtype),
            pltpu.SemaphoreType.DMA((2,)),
        ],
    )(m3, m3, x)


# speedup vs baseline: 1.3110x; 1.0156x over previous
import jax
import jax.numpy as jnp
from jax.experimental import pallas as pl
from jax.experimental.pallas import tpu as pltpu

_CHUNK = 128


def _body(m_ref, mn_ref, x_hbm, o_ref, scratch, sems):
    b = pl.program_id(0)
    nb = pl.num_programs(0)
    _, S, D = scratch.shape
    nc = S // _CHUNK

    length = jnp.sum(m_ref[0, 0, :])
    length_nxt = jnp.sum(mn_ref[0, 0, :])

    def chunk_copy(row, buf, c):
        return pltpu.make_async_copy(
            x_hbm.at[row, pl.ds(c * _CHUNK, _CHUNK), :],
            scratch.at[buf, pl.ds(c * _CHUNK, _CHUNK), :],
            sems.at[buf],
        )

    def issue(row, buf, row_len):
        nch = (row_len + _CHUNK - 1) // _CHUNK

        def st(c, carry):
            @pl.when(c < nch)
            def _():
                chunk_copy(row, buf, c).start()
            return carry

        jax.lax.fori_loop(0, nc, st, 0, unroll=True)

    def wait_row(row, buf, row_len):
        nch = (row_len + _CHUNK - 1) // _CHUNK

        def wt(c, carry):
            @pl.when(c < nch)
            def _():
                chunk_copy(row, buf, c).wait()
            return carry

        jax.lax.fori_loop(0, nc, wt, 0, unroll=True)

    @pl.when(b == 0)
    def _():
        issue(b, 0, length)

    nxt = b + 1

    @pl.when((nxt < nb) & (nxt % 2 == 0))
    def _():
        issue(nxt, 0, length_nxt)

    @pl.when((nxt < nb) & (nxt % 2 == 1))
    def _():
        issue(nxt, 1, length_nxt)

    @pl.when(b % 2 == 0)
    def _():
        wait_row(b, 0, length)

    @pl.when(b % 2 == 1)
    def _():
        wait_row(b, 1, length)

    zeros_c = jnp.zeros((_CHUNK, D), dtype=o_ref.dtype)
    for buf in (0, 1):

        @pl.when(b % 2 == buf)
        def _(buf=buf):
            for c in range(nc):
                lo = c * _CHUNK

                @pl.when(lo < length)
                def _(lo=lo):
                    pos = jax.lax.broadcasted_iota(jnp.int32, (_CHUNK, 1), 0) + lo
                    m_t = m_ref[0, 0, pl.ds(lo, _CHUNK)][:, None]
                    keep = (pos < length) & (m_t > 0)
                    o_ref[0, pl.ds(lo, _CHUNK), :] = jnp.where(
                        keep, scratch[buf, pl.ds(lo, _CHUNK), :], zeros_c
                    )

                @pl.when(lo >= length)
                def _(lo=lo):
                    o_ref[0, pl.ds(lo, _CHUNK), :] = zeros_c


def kernel(x, mask):
    B, S, D = x.shape
    m3 = mask.astype(jnp.int32).reshape(B, 1, S)
    return pl.pallas_call(
        _body,
        grid=(B,),
        in_specs=[
            pl.BlockSpec((1, 1, S), lambda b: (b, 0, 0)),
            pl.BlockSpec((1, 1, S), lambda b: (jnp.minimum(b + 1, B - 1), 0, 0)),
            pl.BlockSpec(memory_space=pl.ANY),
        ],
        out_specs=pl.BlockSpec((1, S, D), lambda b: (b, 0, 0)),
        out_shape=jax.ShapeDtypeStruct((B, S, D), x.dtype),
        scratch_shapes=[
            pltpu.VMEM((2, S, D), x.dtype),
            pltpu.SemaphoreType.DMA((2,)),
        ],
    )(m3, m3, x)
